# baseline (speedup 1.0000x reference)
import functools

import jax
import jax.numpy as jnp
from jax import lax
from jax.experimental import pallas as pl
from jax.experimental.pallas import tpu as pltpu
from jax.experimental.pallas import tpu_sc as plsc

VOCAB = 1000000
DIM = 64
B = 16384
L = 50
EPS = 1e-12

NC = 2
NS = 16
NW = NC * NS
TOTAL = B * L
CHUNK = 128
BT = B // CHUNK
NUNIT = L * BT
NCHUNK = NUNIT // NW
NBUF = 4
DEPTH = NBUF // 2
SEG = DIM // 16


def _rsqrt(x):
    i = plsc.bitcast(x, jnp.int32)
    i = jnp.full((16,), 0x5F3759DF, jnp.int32) - lax.shift_right_arithmetic(
        i, jnp.full((16,), 1, jnp.int32))
    y = plsc.bitcast(i, jnp.float32)
    half = x * 0.5
    for _ in range(3):
        y = y * (1.5 - half * y * y)
    return y


def _ln_chunk(buf, tr, g_v, b_v):
    g = [g_v[pl.ds(k * 16, 16)] for k in range(SEG)]
    b = [b_v[pl.ds(k * 16, 16)] for k in range(SEG)]
    iota16 = lax.iota(jnp.int32, 16)

    @plsc.parallel_loop(0, CHUNK, 1, unroll=4)
    def row(i):
        r = buf.at[i]
        v = [r[pl.ds(k * 16, 16)] for k in range(SEG)]
        t = (v[0] + v[1]) + (v[2] + v[3])
        q = (v[0] * v[0] + v[1] * v[1]) + (v[2] * v[2] + v[3] * v[3])
        ssum = jnp.sum(t)
        qsum = jnp.sum(q)
        mean = jnp.full((16,), ssum, jnp.float32) * (1.0 / DIM)
        var = (jnp.full((16,), qsum, jnp.float32) * (1.0 / DIM)
               - mean * mean + EPS)
        rstd = _rsqrt(var)
        for k in range(SEG):
            r[pl.ds(k * 16, 16)] = (v[k] - mean) * rstd * g[k] + b[k]

    @plsc.parallel_loop(0, CHUNK // 16, 1, unroll=2)
    def blk(j):
        rows = j * 16 + iota16
        for d in range(DIM):
            col = plsc.load_gather(buf, [rows, jnp.full((16,), d, jnp.int32)])
            tr[d // 8, pl.ds((d % 8) * CHUNK + j * 16, 16)] = col


def _body(ids_hbm, table_hbm, g_hbm, b_hbm, out_hbm,
          idx_v, rows_v, tr_v, g_v, b_v, gsems, osems):
    cid = lax.axis_index("c")
    sid = lax.axis_index("s")
    wid = sid * NC + cid
    ubase = wid * NCHUNK

    pltpu.sync_copy(ids_hbm.at[wid], idx_v)
    pltpu.sync_copy(g_hbm, g_v)
    pltpu.sync_copy(b_hbm, b_v)

    def gather_desc(c, r):
        return pltpu.make_async_copy(
            table_hbm.at[idx_v.at[c]], rows_v.at[r], gsems[r])

    def out_descs(c, r):
        u = ubase + c
        ul = lax.shift_right_logical(u, 7)
        ub = lax.bitwise_and(u, 127)
        return pltpu.make_async_copy(
            tr_v.at[r], out_hbm.at[ul, :, ub], osems[r])

    def start_out(c, r):
        out_descs(c, r).start()

    def wait_out(c, r):
        out_descs(c, r).wait()

    for d in range(DEPTH):
        gather_desc(d, d).start()

    def outer(c4, carry):
        for k in range(NBUF):
            c = c4 * NBUF + k
            r = k
            rn = (k + DEPTH) % NBUF

            def prefetch():
                @pl.when(c >= DEPTH)
                def _():
                    wait_out(c - DEPTH, rn)
                gather_desc(c + DEPTH, rn).start()

            if k < NBUF - DEPTH:
                prefetch()
            else:
                @pl.when(c4 <= NCHUNK // NBUF - 2)
                def _():
                    prefetch()

            gather_desc(c, r).wait()
            _ln_chunk(rows_v.at[r], tr_v.at[r], g_v, b_v)
            start_out(c, r)
        return carry

    lax.fori_loop(0, NCHUNK // NBUF, outer, 0)

    for k in range(NBUF):
        c = NCHUNK - NBUF + k
        wait_out(c, k % NBUF)


def kernel(input_ids, table, gamma, beta):
    ids3 = input_ids.T.reshape(NW, NCHUNK, CHUNK).astype(jnp.int32)

    mesh = plsc.VectorSubcoreMesh(core_axis_name="c", subcore_axis_name="s")
    run = pl.kernel(
        _body,
        out_type=jax.ShapeDtypeStruct((L, DIM // 8, BT, 8 * CHUNK),
                                      jnp.float32),
        mesh=mesh,
        compiler_params=pltpu.CompilerParams(
            needs_layout_passes=False, use_tc_tiling_on_sc=False),
        scratch_types=[
            pltpu.VMEM((NCHUNK, CHUNK), jnp.int32),
            pltpu.VMEM((NBUF, CHUNK, DIM), jnp.float32),
            pltpu.VMEM((NBUF, DIM // 8, 8 * CHUNK), jnp.float32),
            pltpu.VMEM((DIM,), jnp.float32),
            pltpu.VMEM((DIM,), jnp.float32),
            [pltpu.SemaphoreType.DMA] * NBUF,
            [pltpu.SemaphoreType.DMA] * NBUF,
        ],
    )
    out = run(ids3, table, gamma, beta)
    out = out.reshape(L, DIM // 8, BT, 8, CHUNK)
    return out.transpose(2, 4, 0, 1, 3).reshape(B, L, DIM)


# speedup vs baseline: 1.2382x; 1.2382x over previous
import functools

import jax
import jax.numpy as jnp
from jax import lax
from jax.experimental import pallas as pl
from jax.experimental.pallas import tpu as pltpu
from jax.experimental.pallas import tpu_sc as plsc

VOCAB = 1000000
DIM = 64
B = 16384
L = 50
EPS = 1e-12

NC = 2
NS = 16
NW = NC * NS
TOTAL = B * L
PER_W = TOTAL // NW
CHUNK = 256
NCHUNK = PER_W // CHUNK
NBUF = 4
DEPTH = NBUF // 2
SEG = DIM // 16


def _rsqrt(x):
    i = plsc.bitcast(x, jnp.int32)
    i = jnp.full((16,), 0x5F3759DF, jnp.int32) - lax.shift_right_arithmetic(
        i, jnp.full((16,), 1, jnp.int32))
    y = plsc.bitcast(i, jnp.float32)
    half = x * 0.5
    for _ in range(3):
        y = y * (1.5 - half * y * y)
    return y


def _ln_chunk(buf, g_v, b_v):
    g = [g_v[pl.ds(k * 16, 16)] for k in range(SEG)]
    b = [b_v[pl.ds(k * 16, 16)] for k in range(SEG)]

    @plsc.parallel_loop(0, CHUNK, 1, unroll=4)
    def row(i):
        r = buf.at[i]
        v = [r[pl.ds(k * 16, 16)] for k in range(SEG)]
        t = (v[0] + v[1]) + (v[2] + v[3])
        q = (v[0] * v[0] + v[1] * v[1]) + (v[2] * v[2] + v[3] * v[3])
        ssum = jnp.sum(t)
        qsum = jnp.sum(q)
        mean = jnp.full((16,), ssum, jnp.float32) * (1.0 / DIM)
        var = (jnp.full((16,), qsum, jnp.float32) * (1.0 / DIM)
               - mean * mean + EPS)
        rstd = _rsqrt(var)
        for k in range(SEG):
            r[pl.ds(k * 16, 16)] = (v[k] - mean) * rstd * g[k] + b[k]


def _body(ids_hbm, table_hbm, g_hbm, b_hbm, out_hbm,
          idx_v, rows_v, g_v, b_v, gsems, osems):
    cid = lax.axis_index("c")
    sid = lax.axis_index("s")
    wid = sid * NC + cid
    base = wid * NCHUNK

    pltpu.sync_copy(ids_hbm.at[wid], idx_v)
    pltpu.sync_copy(g_hbm, g_v)
    pltpu.sync_copy(b_hbm, b_v)

    def gather_desc(c, r):
        return pltpu.make_async_copy(
            table_hbm.at[idx_v.at[c]], rows_v.at[r], gsems[r])

    def out_desc(c, r):
        return pltpu.make_async_copy(
            rows_v.at[r], out_hbm.at[base + c], osems[r])

    for d in range(DEPTH):
        gather_desc(d, d).start()

    def outer(c4, carry):
        for k in range(NBUF):
            c = c4 * NBUF + k
            r = k
            rn = (k + DEPTH) % NBUF

            def prefetch():
                @pl.when(c >= DEPTH)
                def _():
                    out_desc(c - DEPTH, rn).wait()
                gather_desc(c + DEPTH, rn).start()

            if k < NBUF - DEPTH:
                prefetch()
            else:
                @pl.when(c4 <= NCHUNK // NBUF - 2)
                def _():
                    prefetch()

            gather_desc(c, r).wait()
            _ln_chunk(rows_v.at[r], g_v, b_v)
            out_desc(c, r).start()
        return carry

    lax.fori_loop(0, NCHUNK // NBUF, outer, 0)

    for k in range(NBUF):
        c = NCHUNK - NBUF + k
        out_desc(c, k % NBUF).wait()


def kernel(input_ids, table, gamma, beta):
    ids3 = input_ids.reshape(NW, NCHUNK, CHUNK).astype(jnp.int32)

    mesh = plsc.VectorSubcoreMesh(core_axis_name="c", subcore_axis_name="s")
    run = pl.kernel(
        _body,
        out_type=jax.ShapeDtypeStruct((NW * NCHUNK, CHUNK, DIM),
                                      jnp.float32),
        mesh=mesh,
        compiler_params=pltpu.CompilerParams(
            needs_layout_passes=False, use_tc_tiling_on_sc=False),
        scratch_types=[
            pltpu.VMEM((NCHUNK, CHUNK), jnp.int32),
            pltpu.VMEM((NBUF, CHUNK, DIM), jnp.float32),
            pltpu.VMEM((DIM,), jnp.float32),
            pltpu.VMEM((DIM,), jnp.float32),
            [pltpu.SemaphoreType.DMA] * NBUF,
            [pltpu.SemaphoreType.DMA] * NBUF,
        ],
    )
    out = run(ids3, table, gamma, beta)
    return out.reshape(B, L, DIM)
